# baseline (speedup 1.0000x reference)
import functools

import jax
import jax.numpy as jnp
from jax import lax
from jax.experimental import pallas as pl
from jax.experimental.pallas import tpu as pltpu
from jax.experimental.pallas import tpu_sc as plsc

N_NODES = 10000
N_GRAPHS = 64
D = 128
E_TOTAL = 320000

NC = 2
NS = 16
NW = NC * NS
HALF = N_NODES // NC
AGGP = 5040
CHUNK = 80
E_PER_W1 = E_TOTAL // NW
N_CHUNKS1 = E_PER_W1 // CHUNK
E_PER_W2 = E_TOTAL // NS
N_CHUNKS2 = E_PER_W2 // CHUNK
GROUPS = CHUNK // 16
INV_SQRT_D = float(1.0 / (D ** 0.5))

BM = 1000


def _mesh():
    return plsc.VectorSubcoreMesh(
        core_axis_name="c", subcore_axis_name="s",
        num_cores=NC, num_subcores=NS)


def _proj_body(x_ref, w_ref, b_ref, o_ref):
    o_ref[...] = (
        jnp.dot(x_ref[...], w_ref[...], preferred_element_type=jnp.float32)
        + b_ref[...][0][None, :]
    )


def _project(x, wcat, bcat8):
    return pl.pallas_call(
        _proj_body,
        grid=(N_NODES // BM,),
        in_specs=[
            pl.BlockSpec((BM, D), lambda i: (i, 0)),
            pl.BlockSpec((D, 4 * D), lambda i: (0, 0)),
            pl.BlockSpec((8, 4 * D), lambda i: (0, 0)),
        ],
        out_specs=pl.BlockSpec((BM, 4 * D), lambda i: (i, 0)),
        out_shape=jax.ShapeDtypeStruct((N_NODES, 4 * D), jnp.float32),
    )(x, wcat, bcat8)


def _l1_body(q_hbm, k_hbm, src_hbm, dst_hbm, ex_out, den_out,
             src_v, dst_v, qrows, krows, ex_v, zd,
             isem0, isem1, gsem0, gsem1, den_sh):
    cid = lax.axis_index("c")
    sid = lax.axis_index("s")
    wid = sid * NC + cid
    lanes = lax.iota(jnp.int32, 16)
    zero16 = jnp.zeros((16,), jnp.float32)
    isems = (isem0, isem1)
    gsems = (gsem0, gsem1)

    @pl.when(sid == 0)
    def _():
        for g in range(GROUPS):
            zd[pl.ds(g * 16, 16)] = zero16

        def zs(c, carry):
            pltpu.sync_copy(zd, den_sh.at[pl.ds(c * CHUNK, CHUNK)])
            return carry
        lax.fori_loop(0, N_NODES // CHUNK, zs, 0)

    plsc.subcore_barrier()

    base_w = wid * E_PER_W1

    def idx_copies(c, b):
        base = base_w + c * CHUNK
        return (
            pltpu.make_async_copy(
                src_hbm.at[pl.ds(base, CHUNK)], src_v.at[b], isems[b]),
            pltpu.make_async_copy(
                dst_hbm.at[pl.ds(base, CHUNK)], dst_v.at[b], isems[b]),
        )

    def gather_copies(b):
        return (
            pltpu.make_async_copy(q_hbm.at[dst_v.at[b]], qrows.at[b], gsems[b]),
            pltpu.make_async_copy(k_hbm.at[src_v.at[b]], krows.at[b], gsems[b]),
        )

    def do_chunk(c, b):
        o = 1 - b

        @pl.when(c + 1 < N_CHUNKS1)
        def _():
            for cp in idx_copies(c + 1, o):
                cp.wait()
            for cp in gather_copies(o):
                cp.start()

        for cp in gather_copies(b):
            cp.wait()

        bb = jnp.full((16,), b, jnp.int32)
        for g in range(GROUPS):
            le = g * 16 + lanes

            def dotblk(t, acc):
                d0 = t * 8
                for u in range(8):
                    dd = jnp.full((16,), d0 + u, jnp.int32)
                    acc = acc + (plsc.load_gather(qrows, [bb, le, dd])
                                 * plsc.load_gather(krows, [bb, le, dd]))
                return acc
            s = lax.fori_loop(0, D // 8, dotblk, zero16)
            ex_v[pl.ds(g * 16, 16)] = jnp.exp(s * INV_SQRT_D)

        pltpu.sync_copy(ex_v, ex_out.at[pl.ds(base_w + c * CHUNK, CHUNK)])
        pltpu.sync_copy(ex_v, den_sh.at[dst_v.at[b]], add=True)

        @pl.when(c + 2 < N_CHUNKS1)
        def _():
            for cp in idx_copies(c + 2, b):
                cp.start()

    for cp in idx_copies(0, 0):
        cp.start()
    for cp in idx_copies(0, 0):
        cp.wait()
    for cp in gather_copies(0):
        cp.start()
    for cp in idx_copies(1, 1):
        cp.start()
    do_chunk(0, 0)

    def pair_body(p, carry):
        do_chunk(2 * p + 1, 1)
        do_chunk(2 * p + 2, 0)
        return carry
    lax.fori_loop(0, (N_CHUNKS1 - 1) // 2, pair_body, 0)
    plsc.subcore_barrier()

    @pl.when(sid == 0)
    def _():
        pltpu.sync_copy(den_sh, den_out.at[cid])


@functools.cache
def _l1_kernel():
    return functools.partial(
        pl.kernel,
        mesh=_mesh(),
        compiler_params=pltpu.CompilerParams(needs_layout_passes=False),
        out_type=[
            jax.ShapeDtypeStruct((E_TOTAL,), jnp.float32),
            jax.ShapeDtypeStruct((NC, N_NODES), jnp.float32),
        ],
        scratch_types=[
            pltpu.VMEM((2, CHUNK), jnp.int32),
            pltpu.VMEM((2, CHUNK), jnp.int32),
            pltpu.VMEM((2, CHUNK, D), jnp.float32),
            pltpu.VMEM((2, CHUNK, D), jnp.float32),
            pltpu.VMEM((CHUNK,), jnp.float32),
            pltpu.VMEM((CHUNK,), jnp.float32),
            pltpu.SemaphoreType.DMA,
            pltpu.SemaphoreType.DMA,
            pltpu.SemaphoreType.DMA,
            pltpu.SemaphoreType.DMA,
            pltpu.VMEM_SHARED((N_NODES,), jnp.float32),
        ],
    )(_l1_body)


def _l2_body(v_hbm, src_hbm, dst_hbm, ex_hbm, agg_out,
             src_v, dst_v, exb_v, vrows, vbuf, ldst_v, zrow,
             isem0, isem1, gsem0, gsem1, agg_sh):
    cid = lax.axis_index("c")
    sid = lax.axis_index("s")
    lanes = lax.iota(jnp.int32, 16)
    zero16 = jnp.zeros((16,), jnp.float32)
    isems = (isem0, isem1)
    gsems = (gsem0, gsem1)
    node_base = cid * HALF

    @pl.when(sid == 0)
    def _():
        def zr(r, carry):
            for j in range(8):
                zrow[r, pl.ds(j * 16, 16)] = zero16
            return carry
        lax.fori_loop(0, CHUNK, zr, 0)

        def zs(c, carry):
            pltpu.sync_copy(zrow, agg_sh.at[pl.ds(c * CHUNK, CHUNK)])
            return carry
        lax.fori_loop(0, AGGP // CHUNK, zs, 0)

    plsc.subcore_barrier()

    base_w = sid * E_PER_W2

    def idx_copies(c, b):
        base = base_w + c * CHUNK
        return (
            pltpu.make_async_copy(
                src_hbm.at[pl.ds(base, CHUNK)], src_v.at[b], isems[b]),
            pltpu.make_async_copy(
                dst_hbm.at[pl.ds(base, CHUNK)], dst_v.at[b], isems[b]),
            pltpu.make_async_copy(
                ex_hbm.at[pl.ds(base, CHUNK)], exb_v.at[b], isems[b]),
        )

    def gather_copies(b):
        return (
            pltpu.make_async_copy(v_hbm.at[src_v.at[b]], vrows.at[b], gsems[b]),
        )

    def do_chunk(c, b):
        o = 1 - b

        @pl.when(c + 1 < N_CHUNKS2)
        def _():
            for cp in idx_copies(c + 1, o):
                cp.wait()
            for cp in gather_copies(o):
                cp.start()

        for cp in gather_copies(b):
            cp.wait()

        bb = jnp.full((16,), b, jnp.int32)
        for g in range(GROUPS):
            le = g * 16 + lanes
            dsl = dst_v[b, pl.ds(g * 16, 16)]
            loc = dsl - node_base
            loc = jnp.where((loc >= 0) & (loc < HALF), loc, HALF)
            ldst_v[pl.ds(g * 16, 16)] = loc
            exv = exb_v[b, pl.ds(g * 16, 16)]

            def vsblk(t, carry2):
                d0 = t * 8
                for u in range(8):
                    dd = jnp.full((16,), d0 + u, jnp.int32)
                    plsc.store_scatter(
                        vbuf, [le, dd],
                        plsc.load_gather(vrows, [bb, le, dd]) * exv)
                return carry2
            lax.fori_loop(0, D // 8, vsblk, 0)

        pltpu.sync_copy(vbuf, agg_sh.at[ldst_v], add=True)

        @pl.when(c + 2 < N_CHUNKS2)
        def _():
            for cp in idx_copies(c + 2, b):
                cp.start()

    for cp in idx_copies(0, 0):
        cp.start()
    for cp in idx_copies(0, 0):
        cp.wait()
    for cp in gather_copies(0):
        cp.start()
    for cp in idx_copies(1, 1):
        cp.start()
    do_chunk(0, 0)

    def pair_body(p, carry):
        do_chunk(2 * p + 1, 1)
        do_chunk(2 * p + 2, 0)
        return carry
    lax.fori_loop(0, (N_CHUNKS2 - 1) // 2, pair_body, 0)
    plsc.subcore_barrier()

    @pl.when(sid == 0)
    def _():
        pltpu.sync_copy(agg_sh.at[pl.ds(0, HALF)],
                        agg_out.at[pl.ds(node_base, HALF)])


@functools.cache
def _l2_kernel():
    return functools.partial(
        pl.kernel,
        mesh=_mesh(),
        compiler_params=pltpu.CompilerParams(needs_layout_passes=False),
        out_type=[
            jax.ShapeDtypeStruct((N_NODES, D), jnp.float32),
        ],
        scratch_types=[
            pltpu.VMEM((2, CHUNK), jnp.int32),
            pltpu.VMEM((2, CHUNK), jnp.int32),
            pltpu.VMEM((2, CHUNK), jnp.float32),
            pltpu.VMEM((2, CHUNK, D), jnp.float32),
            pltpu.VMEM((CHUNK, D), jnp.float32),
            pltpu.VMEM((CHUNK,), jnp.int32),
            pltpu.VMEM((CHUNK, D), jnp.float32),
            pltpu.SemaphoreType.DMA,
            pltpu.SemaphoreType.DMA,
            pltpu.SemaphoreType.DMA,
            pltpu.SemaphoreType.DMA,
            pltpu.VMEM_SHARED((AGGP, D), jnp.float32),
        ],
    )(_l2_body)


def _pool_body(agg_ref, denA_ref, denB_ref, skip_ref, batch_ref,
               o_ref, acc, cnt):
    i = pl.program_id(0)

    @pl.when(i == 0)
    def _():
        acc[...] = jnp.zeros_like(acc)
        cnt[...] = jnp.zeros_like(cnt)

    den = denA_ref[0, 0] + denB_ref[0, 0]
    rows = agg_ref[...] / (den[:, None] + 1e-16) + skip_ref[...]
    rows = jnp.maximum(rows, 0.0)
    b = batch_ref[0, 0]
    oh = (b[None, :] == lax.broadcasted_iota(jnp.int32, (N_GRAPHS, BM), 0)
          ).astype(jnp.float32)
    acc[...] += jnp.dot(oh, rows, preferred_element_type=jnp.float32)
    cnt[...] += jnp.sum(oh, axis=1)[:, None]

    @pl.when(i == (N_NODES // BM) - 1)
    def _():
        o_ref[...] = acc[...] / jnp.maximum(cnt[...], 1.0)


def _pool(agg, den3A, den3B, skip, batch3):
    return pl.pallas_call(
        _pool_body,
        grid=(N_NODES // BM,),
        in_specs=[
            pl.BlockSpec((BM, D), lambda i: (i, 0)),
            pl.BlockSpec((1, 1, BM), lambda i: (i, 0, 0)),
            pl.BlockSpec((1, 1, BM), lambda i: (i, 0, 0)),
            pl.BlockSpec((BM, D), lambda i: (i, 0)),
            pl.BlockSpec((1, 1, BM), lambda i: (i, 0, 0)),
        ],
        out_specs=pl.BlockSpec((N_GRAPHS, D), lambda i: (0, 0)),
        out_shape=jax.ShapeDtypeStruct((N_GRAPHS, D), jnp.float32),
        scratch_shapes=[
            pltpu.VMEM((N_GRAPHS, D), jnp.float32),
            pltpu.VMEM((N_GRAPHS, D), jnp.float32),
        ],
    )(agg, den3A, den3B, skip, batch3)


def kernel(x, edge_index, batch, Wq, bq, Wk, bk, Wv, bv, Ws, bs):
    wcat = jnp.concatenate([Wq, Wk, Wv, Ws], axis=1)
    bcat8 = jnp.tile(jnp.concatenate([bq, bk, bv, bs])[None, :], (8, 1))
    proj = _project(x, wcat, bcat8)
    q = proj[:, 0:D]
    k = proj[:, D:2 * D]
    v = proj[:, 2 * D:3 * D]
    skip = proj[:, 3 * D:4 * D]

    src = edge_index[0]
    dst = edge_index[1]
    ex, den2 = _l1_kernel()(q, k, src, dst)
    agg, = _l2_kernel()(v, src, dst, ex)

    nb = N_NODES // BM
    den3A = den2[0].reshape(nb, 1, BM)
    den3B = den2[1].reshape(nb, 1, BM)
    batch3 = batch.reshape(nb, 1, BM)
    return _pool(agg, den3A, den3B, skip, batch3)


# speedup vs baseline: 1.5008x; 1.5008x over previous
import functools

import jax
import jax.numpy as jnp
from jax import lax
from jax.experimental import pallas as pl
from jax.experimental.pallas import tpu as pltpu
from jax.experimental.pallas import tpu_sc as plsc

N_NODES = 10000
N_GRAPHS = 64
D = 128
E_TOTAL = 320000

NC = 2
NS = 16
NW = NC * NS
HALF = N_NODES // NC
AGGP = 5040
CHUNK = 80
E_PER_W1 = E_TOTAL // NW
N_CHUNKS1 = E_PER_W1 // CHUNK
E_PER_W2 = E_TOTAL // NS
N_CHUNKS2 = E_PER_W2 // CHUNK
GROUPS = CHUNK // 16
INV_SQRT_D = float(1.0 / (D ** 0.5))

BM = 1000


def _mesh():
    return plsc.VectorSubcoreMesh(
        core_axis_name="c", subcore_axis_name="s",
        num_cores=NC, num_subcores=NS)


def _proj_body(x_ref, w_ref, b_ref, o_ref):
    o_ref[...] = (
        jnp.dot(x_ref[...], w_ref[...], preferred_element_type=jnp.float32)
        + b_ref[...][0][None, :]
    )


def _project(x, wcat, bcat8):
    return pl.pallas_call(
        _proj_body,
        grid=(N_NODES // BM,),
        in_specs=[
            pl.BlockSpec((BM, D), lambda i: (i, 0)),
            pl.BlockSpec((D, 4 * D), lambda i: (0, 0)),
            pl.BlockSpec((8, 4 * D), lambda i: (0, 0)),
        ],
        out_specs=pl.BlockSpec((BM, 4 * D), lambda i: (i, 0)),
        out_shape=jax.ShapeDtypeStruct((N_NODES, 4 * D), jnp.float32),
    )(x, wcat, bcat8)


def _l1_body(q_hbm, k_hbm, src_hbm, dst_hbm, ex_out, den_out,
             src_v, dst_v, qrows, krows, ex_v, zd,
             isem0, isem1, gsem0, gsem1, ssem0, ssem1, den_sh):
    ssems = (ssem0, ssem1)
    cid = lax.axis_index("c")
    sid = lax.axis_index("s")
    wid = sid * NC + cid
    lanes = lax.iota(jnp.int32, 16)
    zero16 = jnp.zeros((16,), jnp.float32)
    isems = (isem0, isem1)
    gsems = (gsem0, gsem1)

    @pl.when(sid == 0)
    def _():
        for g in range(GROUPS):
            zd[pl.ds(g * 16, 16)] = zero16

        def zs(c, carry):
            pltpu.sync_copy(zd, den_sh.at[pl.ds(c * CHUNK, CHUNK)])
            return carry
        lax.fori_loop(0, N_NODES // CHUNK, zs, 0)

    plsc.subcore_barrier()

    base_w = wid * E_PER_W1

    def idx_copies(c, b):
        base = base_w + c * CHUNK
        return (
            pltpu.make_async_copy(
                src_hbm.at[pl.ds(base, CHUNK)], src_v.at[b], isems[b]),
            pltpu.make_async_copy(
                dst_hbm.at[pl.ds(base, CHUNK)], dst_v.at[b], isems[b]),
        )

    def gather_copies(b):
        return (
            pltpu.make_async_copy(q_hbm.at[dst_v.at[b]], qrows.at[b], gsems[b]),
            pltpu.make_async_copy(k_hbm.at[src_v.at[b]], krows.at[b], gsems[b]),
        )

    def ex_store(c, b):
        return pltpu.make_async_copy(
            ex_v.at[b], ex_out.at[pl.ds(base_w + c * CHUNK, CHUNK)], ssems[b])

    def do_chunk(c, b):
        o = 1 - b

        @pl.when(c >= 2)
        def _():
            ex_store(c - 2, b).wait()

        @pl.when(c + 1 < N_CHUNKS1)
        def _():
            for cp in idx_copies(c + 1, o):
                cp.wait()
            for cp in gather_copies(o):
                cp.start()

        for cp in gather_copies(b):
            cp.wait()

        bb = jnp.full((16,), b, jnp.int32)
        for g in range(GROUPS):
            le = g * 16 + lanes

            def dotblk(t, acc):
                d0 = t * 8
                for u in range(8):
                    dd = jnp.full((16,), d0 + u, jnp.int32)
                    acc = acc + (plsc.load_gather(qrows, [bb, le, dd])
                                 * plsc.load_gather(krows, [bb, le, dd]))
                return acc
            s = lax.fori_loop(0, D // 8, dotblk, zero16)
            ex_v[b, pl.ds(g * 16, 16)] = jnp.exp(s * INV_SQRT_D)

        ex_store(c, b).start()
        pltpu.sync_copy(ex_v.at[b], den_sh.at[dst_v.at[b]], add=True)

        @pl.when(c + 2 < N_CHUNKS1)
        def _():
            for cp in idx_copies(c + 2, b):
                cp.start()

    for cp in idx_copies(0, 0):
        cp.start()
    for cp in idx_copies(0, 0):
        cp.wait()
    for cp in gather_copies(0):
        cp.start()
    for cp in idx_copies(1, 1):
        cp.start()
    do_chunk(0, 0)

    def pair_body(p, carry):
        do_chunk(2 * p + 1, 1)
        do_chunk(2 * p + 2, 0)
        return carry
    lax.fori_loop(0, (N_CHUNKS1 - 1) // 2, pair_body, 0)
    ex_store(N_CHUNKS1 - 2, (N_CHUNKS1 - 2) & 1).wait()
    ex_store(N_CHUNKS1 - 1, (N_CHUNKS1 - 1) & 1).wait()
    plsc.subcore_barrier()

    @pl.when(sid == 0)
    def _():
        pltpu.sync_copy(den_sh, den_out.at[cid])


@functools.cache
def _l1_kernel():
    return functools.partial(
        pl.kernel,
        mesh=_mesh(),
        compiler_params=pltpu.CompilerParams(needs_layout_passes=False),
        out_type=[
            jax.ShapeDtypeStruct((E_TOTAL,), jnp.float32),
            jax.ShapeDtypeStruct((NC, N_NODES), jnp.float32),
        ],
        scratch_types=[
            pltpu.VMEM((2, CHUNK), jnp.int32),
            pltpu.VMEM((2, CHUNK), jnp.int32),
            pltpu.VMEM((2, CHUNK, D), jnp.float32),
            pltpu.VMEM((2, CHUNK, D), jnp.float32),
            pltpu.VMEM((2, CHUNK), jnp.float32),
            pltpu.VMEM((CHUNK,), jnp.float32),
            pltpu.SemaphoreType.DMA,
            pltpu.SemaphoreType.DMA,
            pltpu.SemaphoreType.DMA,
            pltpu.SemaphoreType.DMA,
            pltpu.SemaphoreType.DMA,
            pltpu.SemaphoreType.DMA,
            pltpu.VMEM_SHARED((N_NODES,), jnp.float32),
        ],
    )(_l1_body)


def _l2_body(vt_hbm, src_hbm, dst_hbm, ex_hbm, agg_out,
             src_v, dst_v, exb_v, srcm_v, vrows, vbuf, zrow,
             isem0, isem1, gsem0, gsem1, agg_sh):
    cid = lax.axis_index("c")
    sid = lax.axis_index("s")
    lanes = lax.iota(jnp.int32, 16)
    zero16 = jnp.zeros((16,), jnp.float32)
    isems = (isem0, isem1)
    gsems = (gsem0, gsem1)
    D2 = D // 2

    @pl.when(sid == 0)
    def _():
        def zr(r, carry):
            for j in range(4):
                zrow[r, pl.ds(j * 16, 16)] = zero16
            return carry
        lax.fori_loop(0, CHUNK, zr, 0)

        def zs(c, carry):
            pltpu.sync_copy(zrow, agg_sh.at[pl.ds(c * CHUNK, CHUNK)])
            return carry
        lax.fori_loop(0, N_NODES // CHUNK, zs, 0)

    plsc.subcore_barrier()

    base_w = sid * E_PER_W2

    def idx_copies(c, b):
        base = base_w + c * CHUNK
        return (
            pltpu.make_async_copy(
                src_hbm.at[pl.ds(base, CHUNK)], src_v.at[b], isems[b]),
            pltpu.make_async_copy(
                dst_hbm.at[pl.ds(base, CHUNK)], dst_v.at[b], isems[b]),
            pltpu.make_async_copy(
                ex_hbm.at[pl.ds(base, CHUNK)], exb_v.at[b], isems[b]),
        )

    def fill_srcm(b):
        off = cid * N_NODES
        for g in range(GROUPS):
            srcm_v[b, pl.ds(g * 16, 16)] = src_v[b, pl.ds(g * 16, 16)] + off

    def gather_copies(b):
        return (
            pltpu.make_async_copy(
                vt_hbm.at[srcm_v.at[b]], vrows.at[b], gsems[b]),
        )

    def do_chunk(c, b):
        o = 1 - b

        @pl.when(c + 1 < N_CHUNKS2)
        def _():
            for cp in idx_copies(c + 1, o):
                cp.wait()
            fill_srcm(o)
            for cp in gather_copies(o):
                cp.start()

        for cp in gather_copies(b):
            cp.wait()

        bb = jnp.full((16,), b, jnp.int32)
        for g in range(GROUPS):
            le = g * 16 + lanes
            exv = exb_v[b, pl.ds(g * 16, 16)]

            def vsblk(t, carry2):
                d0 = t * 8
                for u in range(8):
                    dd = jnp.full((16,), d0 + u, jnp.int32)
                    plsc.store_scatter(
                        vbuf, [le, dd],
                        plsc.load_gather(vrows, [bb, le, dd]) * exv)
                return carry2
            lax.fori_loop(0, D2 // 8, vsblk, 0)

        pltpu.sync_copy(vbuf, agg_sh.at[dst_v.at[b]], add=True)

        @pl.when(c + 2 < N_CHUNKS2)
        def _():
            for cp in idx_copies(c + 2, b):
                cp.start()

    for cp in idx_copies(0, 0):
        cp.start()
    for cp in idx_copies(0, 0):
        cp.wait()
    fill_srcm(0)
    for cp in gather_copies(0):
        cp.start()
    for cp in idx_copies(1, 1):
        cp.start()
    do_chunk(0, 0)

    def pair_body(p, carry):
        do_chunk(2 * p + 1, 1)
        do_chunk(2 * p + 2, 0)
        return carry
    lax.fori_loop(0, (N_CHUNKS2 - 1) // 2, pair_body, 0)
    plsc.subcore_barrier()

    @pl.when(sid == 0)
    def _():
        pltpu.sync_copy(agg_sh, agg_out.at[cid])


@functools.cache
def _l2_kernel():
    return functools.partial(
        pl.kernel,
        mesh=_mesh(),
        compiler_params=pltpu.CompilerParams(
            needs_layout_passes=False, use_tc_tiling_on_sc=False),
        out_type=[
            jax.ShapeDtypeStruct((NC, N_NODES, D // 2), jnp.float32),
        ],
        scratch_types=[
            pltpu.VMEM((2, CHUNK), jnp.int32),
            pltpu.VMEM((2, CHUNK), jnp.int32),
            pltpu.VMEM((2, CHUNK), jnp.float32),
            pltpu.VMEM((2, CHUNK), jnp.int32),
            pltpu.VMEM((2, CHUNK, D // 2), jnp.float32),
            pltpu.VMEM((CHUNK, D // 2), jnp.float32),
            pltpu.VMEM((CHUNK, D // 2), jnp.float32),
            pltpu.SemaphoreType.DMA,
            pltpu.SemaphoreType.DMA,
            pltpu.SemaphoreType.DMA,
            pltpu.SemaphoreType.DMA,
            pltpu.VMEM_SHARED((N_NODES, D // 2), jnp.float32),
        ],
    )(_l2_body)


def _pool_body(agg_ref, denA_ref, denB_ref, skip_ref, batch_ref,
               o_ref, acc, cnt):
    i = pl.program_id(0)

    @pl.when(i == 0)
    def _():
        acc[...] = jnp.zeros_like(acc)
        cnt[...] = jnp.zeros_like(cnt)

    den = denA_ref[0, 0] + denB_ref[0, 0]
    rows = agg_ref[...] / (den[:, None] + 1e-16) + skip_ref[...]
    rows = jnp.maximum(rows, 0.0)
    b = batch_ref[0, 0]
    oh = (b[None, :] == lax.broadcasted_iota(jnp.int32, (N_GRAPHS, BM), 0)
          ).astype(jnp.float32)
    acc[...] += jnp.dot(oh, rows, preferred_element_type=jnp.float32)
    cnt[...] += jnp.sum(oh, axis=1)[:, None]

    @pl.when(i == (N_NODES // BM) - 1)
    def _():
        o_ref[...] = acc[...] / jnp.maximum(cnt[...], 1.0)


def _pool(agg, den3A, den3B, skip, batch3):
    return pl.pallas_call(
        _pool_body,
        grid=(N_NODES // BM,),
        in_specs=[
            pl.BlockSpec((BM, D), lambda i: (i, 0)),
            pl.BlockSpec((1, 1, BM), lambda i: (i, 0, 0)),
            pl.BlockSpec((1, 1, BM), lambda i: (i, 0, 0)),
            pl.BlockSpec((BM, D), lambda i: (i, 0)),
            pl.BlockSpec((1, 1, BM), lambda i: (i, 0, 0)),
        ],
        out_specs=pl.BlockSpec((N_GRAPHS, D), lambda i: (0, 0)),
        out_shape=jax.ShapeDtypeStruct((N_GRAPHS, D), jnp.float32),
        scratch_shapes=[
            pltpu.VMEM((N_GRAPHS, D), jnp.float32),
            pltpu.VMEM((N_GRAPHS, D), jnp.float32),
        ],
    )(agg, den3A, den3B, skip, batch3)


def kernel(x, edge_index, batch, Wq, bq, Wk, bk, Wv, bv, Ws, bs):
    wcat = jnp.concatenate([Wq, Wk, Wv, Ws], axis=1)
    bcat8 = jnp.tile(jnp.concatenate([bq, bk, bv, bs])[None, :], (8, 1))
    proj = _project(x, wcat, bcat8)
    q = proj[:, 0:D]
    k = proj[:, D:2 * D]
    v = proj[:, 2 * D:3 * D]
    skip = proj[:, 3 * D:4 * D]

    src = edge_index[0]
    dst = edge_index[1]
    ex, den2 = _l1_kernel()(q, k, src, dst)
    vt = jnp.concatenate([v[:, :D // 2], v[:, D // 2:]], axis=0)
    agg2, = _l2_kernel()(vt, src, dst, ex)
    agg = jnp.concatenate([agg2[0], agg2[1]], axis=1)

    nb = N_NODES // BM
    den3A = den2[0].reshape(nb, 1, BM)
    den3B = den2[1].reshape(nb, 1, BM)
    batch3 = batch.reshape(nb, 1, BM)
    return _pool(agg, den3A, den3B, skip, batch3)
